# baseline (speedup 1.0000x reference)
import jax
import jax.numpy as jnp
import numpy as np
from jax.experimental import pallas as pl
from jax.experimental.pallas import tpu as pltpu

_H = 12
_DH = 64
_EPS = 1e-06
_Z_LOSS_COEF = 0.001
_TOPK = 2


def _qkv_kernel(x_ref, ln_ref, wq_ref, wk_ref, wv_ref, q_ref, k_ref, v_ref):
    x = x_ref[...]
    var = jnp.mean(x * x, axis=1, keepdims=True)
    xn = (ln_ref[...] * (x * jax.lax.rsqrt(var + _EPS))).astype(jnp.bfloat16)
    dims = (((1,), (1,)), ((), ()))
    q_ref[...] = jax.lax.dot_general(xn, wq_ref[...].astype(jnp.bfloat16),
                                     dims, preferred_element_type=jnp.float32)
    k_ref[...] = jax.lax.dot_general(xn, wk_ref[...].astype(jnp.bfloat16),
                                     dims, preferred_element_type=jnp.float32)
    v_ref[...] = jax.lax.dot_general(xn, wv_ref[...].astype(jnp.bfloat16),
                                     dims, preferred_element_type=jnp.float32)


def _attn_kernel(q_ref, k_ref, v_ref, cos_ref, sin_ref, o_ref):
    q = q_ref[0]
    k = k_ref[0]
    v = v_ref[0]
    cos = cos_ref[...]
    sin = sin_ref[...]
    half = _DH // 2
    q_rot = jnp.concatenate([-q[:, half:], q[:, :half]], axis=1)
    k_rot = jnp.concatenate([-k[:, half:], k[:, :half]], axis=1)
    qr = ((q * cos + q_rot * sin) * (1.0 / 8.0)).astype(jnp.bfloat16)
    kr = (k * cos + k_rot * sin).astype(jnp.bfloat16)
    scores = jax.lax.dot_general(qr, kr, (((1,), (1,)), ((), ())),
                                 preferred_element_type=jnp.float32)
    m = jnp.max(scores, axis=1, keepdims=True)
    p = jnp.exp(scores - m)
    s = jnp.sum(p, axis=1, keepdims=True)
    o = jnp.dot(p.astype(jnp.bfloat16), v.astype(jnp.bfloat16),
                preferred_element_type=jnp.float32)
    o_ref[0] = o / s


def _post_attn_kernel(a_ref, res_ref, wo_ref, ln_ref, wg_ref,
                      h_ref, hn_ref, comb_ref, zacc_ref):
    a = a_ref[...].astype(jnp.bfloat16)
    h = res_ref[...] + jax.lax.dot_general(
        a, wo_ref[...].astype(jnp.bfloat16), (((1,), (1,)), ((), ())),
        preferred_element_type=jnp.float32)
    h_ref[...] = h
    var = jnp.mean(h * h, axis=1, keepdims=True)
    hn = ln_ref[...] * (h * jax.lax.rsqrt(var + _EPS))
    hn_ref[...] = hn
    logits = jax.lax.dot_general(hn, wg_ref[...], (((1,), (1,)), ((), ())),
                                 preferred_element_type=jnp.float32)
    e = logits.shape[1]
    lm = jnp.max(logits, axis=1, keepdims=True)
    ex = jnp.exp(logits - lm)
    p = ex / jnp.sum(ex, axis=1, keepdims=True)
    iota = jax.lax.broadcasted_iota(jnp.int32, p.shape, 1)
    m1 = jnp.max(p, axis=1, keepdims=True)
    i1 = jnp.min(jnp.where(p == m1, iota, e), axis=1, keepdims=True)
    sel1 = iota == i1
    p2 = jnp.where(sel1, -1.0, p)
    m2 = jnp.max(p2, axis=1, keepdims=True)
    i2 = jnp.min(jnp.where(p2 == m2, iota, e), axis=1, keepdims=True)
    sel2 = iota == i2
    denom = m1 + m2
    comb_ref[...] = (jnp.where(sel1, m1, 0.0) +
                     jnp.where(sel2, m2, 0.0)) / denom
    z = lm + jnp.log(jnp.sum(ex, axis=1, keepdims=True))
    tile_sum = jnp.sum(z * z, axis=0, keepdims=True)

    @pl.when(pl.program_id(0) == 0)
    def _():
        zacc_ref[...] = jnp.zeros_like(zacc_ref)

    zacc_ref[...] += tile_sum


def _moe_kernel(hn_ref, h_ref, comb_ref, wg_ref, wu_ref, wd_ref, o_ref,
                acc_ref):
    e = pl.program_id(0)
    f = pl.program_id(1)
    t = pl.program_id(2)
    ne = pl.num_programs(0)
    nf = pl.num_programs(1)
    tm = hn_ref.shape[0]
    xs = hn_ref[...].astype(jnp.bfloat16)
    dims = (((1,), (1,)), ((), ()))
    g = jax.lax.dot_general(xs, wg_ref[0].astype(jnp.bfloat16), dims,
                            preferred_element_type=jnp.float32)
    u = jax.lax.dot_general(xs, wu_ref[0].astype(jnp.bfloat16), dims,
                            preferred_element_type=jnp.float32)
    a = (jax.nn.silu(g) * u).astype(jnp.bfloat16)
    y = jax.lax.dot_general(a, wd_ref[0].astype(jnp.bfloat16), dims,
                            preferred_element_type=jnp.float32)
    onehot = (jax.lax.broadcasted_iota(jnp.int32, (comb_ref.shape[1], 1), 0)
              == e).astype(jnp.float32)
    w = jnp.dot(comb_ref[...], onehot, preferred_element_type=jnp.float32)
    contrib = w * y
    row = pl.ds(t * tm, tm)

    @pl.when((e == 0) & (f == 0))
    def _():
        acc_ref[row, :] = h_ref[...] + contrib

    @pl.when((e > 0) | (f > 0))
    def _():
        acc_ref[row, :] += contrib

    @pl.when((e == ne - 1) & (f == nf - 1))
    def _():
        o_ref[...] = acc_ref[row, :]


@jax.jit
def kernel(hidden_states, ln1_w, ln2_w, Wq, Wk, Wv, Wo, Wg, We_gate, We_up, We_down):
    B, S, D = hidden_states.shape
    E, FF, _ = We_gate.shape
    T = B * S
    TM = 512
    x = hidden_states.reshape(T, D)

    f32 = jnp.float32
    bf16 = jnp.bfloat16
    ln1 = ln1_w.reshape(1, D)
    ln2 = ln2_w.reshape(1, D)

    q, k, v = pl.pallas_call(
        _qkv_kernel,
        grid=(T // TM,),
        in_specs=[
            pl.BlockSpec((TM, D), lambda t: (t, 0)),
            pl.BlockSpec((1, D), lambda t: (0, 0)),
            pl.BlockSpec((D, D), lambda t: (0, 0)),
            pl.BlockSpec((D, D), lambda t: (0, 0)),
            pl.BlockSpec((D, D), lambda t: (0, 0)),
        ],
        out_specs=[
            pl.BlockSpec((TM, D), lambda t: (t, 0)),
            pl.BlockSpec((TM, D), lambda t: (t, 0)),
            pl.BlockSpec((TM, D), lambda t: (t, 0)),
        ],
        out_shape=[jax.ShapeDtypeStruct((T, D), f32)] * 3,
    )(x, ln1, Wq, Wk, Wv)

    def to_heads(t):
        return (t.reshape(B, S, _H, _DH).transpose(0, 2, 1, 3)
                .reshape(B * _H, S, _DH))

    qh, kh, vh = to_heads(q), to_heads(k), to_heads(v)

    inv_freq = 1.0 / (10000.0 ** (jnp.arange(0, _DH, 2, dtype=f32) / _DH))
    t_pos = jnp.arange(S, dtype=f32)
    freqs = jnp.outer(t_pos, inv_freq)
    emb = jnp.concatenate([freqs, freqs], axis=-1)
    cos = jnp.cos(emb)
    sin = jnp.sin(emb)

    attn = pl.pallas_call(
        _attn_kernel,
        grid=(B * _H,),
        in_specs=[
            pl.BlockSpec((1, S, _DH), lambda i: (i, 0, 0)),
            pl.BlockSpec((1, S, _DH), lambda i: (i, 0, 0)),
            pl.BlockSpec((1, S, _DH), lambda i: (i, 0, 0)),
            pl.BlockSpec((S, _DH), lambda i: (0, 0)),
            pl.BlockSpec((S, _DH), lambda i: (0, 0)),
        ],
        out_specs=pl.BlockSpec((1, S, _DH), lambda i: (i, 0, 0)),
        out_shape=jax.ShapeDtypeStruct((B * _H, S, _DH), f32),
    )(qh, kh, vh, cos, sin)

    attn_flat = (attn.reshape(B, _H, S, _DH).transpose(0, 2, 1, 3)
                 .reshape(T, D))

    h, hn, comb, zacc = pl.pallas_call(
        _post_attn_kernel,
        grid=(T // TM,),
        in_specs=[
            pl.BlockSpec((TM, D), lambda t: (t, 0)),
            pl.BlockSpec((TM, D), lambda t: (t, 0)),
            pl.BlockSpec((D, D), lambda t: (0, 0)),
            pl.BlockSpec((1, D), lambda t: (0, 0)),
            pl.BlockSpec((E, D), lambda t: (0, 0)),
        ],
        out_specs=[
            pl.BlockSpec((TM, D), lambda t: (t, 0)),
            pl.BlockSpec((TM, D), lambda t: (t, 0)),
            pl.BlockSpec((TM, E), lambda t: (t, 0)),
            pl.BlockSpec((1, 1), lambda t: (0, 0)),
        ],
        out_shape=[
            jax.ShapeDtypeStruct((T, D), f32),
            jax.ShapeDtypeStruct((T, D), f32),
            jax.ShapeDtypeStruct((T, E), f32),
            jax.ShapeDtypeStruct((1, 1), f32),
        ],
    )(attn_flat, x, Wo, ln2, Wg)

    aux_loss = _Z_LOSS_COEF * zacc[0, 0] / T

    TM2 = 512
    FH = FF // 2
    out = pl.pallas_call(
        _moe_kernel,
        grid=(E, 2, T // TM2),
        in_specs=[
            pl.BlockSpec((TM2, D), lambda e, f, t: (t, 0)),
            pl.BlockSpec((TM2, D), lambda e, f, t: (t, 0)),
            pl.BlockSpec((TM2, E), lambda e, f, t: (t, 0)),
            pl.BlockSpec((1, FH, D), lambda e, f, t: (e, f, 0)),
            pl.BlockSpec((1, FH, D), lambda e, f, t: (e, f, 0)),
            pl.BlockSpec((1, D, FH), lambda e, f, t: (e, 0, f)),
        ],
        out_specs=pl.BlockSpec((TM2, D), lambda e, f, t: (t, 0)),
        out_shape=jax.ShapeDtypeStruct((T, D), f32),
        scratch_shapes=[pltpu.VMEM((T, D), jnp.float32)],
    )(hn, h, comb, We_gate, We_up, We_down)

    return out.reshape(B, S, D), aux_loss


# speedup vs baseline: 1.0658x; 1.0658x over previous
import jax
import jax.numpy as jnp
import numpy as np
from jax.experimental import pallas as pl
from jax.experimental.pallas import tpu as pltpu

_H = 12
_DH = 64
_EPS = 1e-06
_Z_LOSS_COEF = 0.001
_TOPK = 2


def _qkv_kernel(x_ref, ln_ref, wq_ref, wk_ref, wv_ref, q_ref, k_ref, v_ref):
    x = x_ref[...]
    var = jnp.mean(x * x, axis=1, keepdims=True)
    xn = (ln_ref[...] * (x * jax.lax.rsqrt(var + _EPS))).astype(jnp.bfloat16)
    dims = (((1,), (1,)), ((), ()))
    q_ref[...] = jax.lax.dot_general(xn, wq_ref[...].astype(jnp.bfloat16),
                                     dims, preferred_element_type=jnp.float32)
    k_ref[...] = jax.lax.dot_general(xn, wk_ref[...].astype(jnp.bfloat16),
                                     dims, preferred_element_type=jnp.float32)
    v_ref[...] = jax.lax.dot_general(xn, wv_ref[...].astype(jnp.bfloat16),
                                     dims, preferred_element_type=jnp.float32)


def _attn_kernel(q_ref, k_ref, v_ref, cos_ref, sin_ref, o_ref):
    q = q_ref[0]
    k = k_ref[0]
    v = v_ref[0]
    cos = cos_ref[...]
    sin = sin_ref[...]
    half = _DH // 2
    q_rot = jnp.concatenate([-q[:, half:], q[:, :half]], axis=1)
    k_rot = jnp.concatenate([-k[:, half:], k[:, :half]], axis=1)
    qr = ((q * cos + q_rot * sin) * (1.0 / 8.0)).astype(jnp.bfloat16)
    kr = (k * cos + k_rot * sin).astype(jnp.bfloat16)
    scores = jax.lax.dot_general(qr, kr, (((1,), (1,)), ((), ())),
                                 preferred_element_type=jnp.float32)
    m = jnp.max(scores, axis=1, keepdims=True)
    p = jnp.exp(scores - m)
    s = jnp.sum(p, axis=1, keepdims=True)
    o = jnp.dot(p.astype(jnp.bfloat16), v.astype(jnp.bfloat16),
                preferred_element_type=jnp.float32)
    o_ref[0] = o / s


def _post_attn_kernel(a_ref, res_ref, wo_ref, ln_ref, wg_ref,
                      h_ref, hn_ref, comb_ref, zacc_ref):
    a = a_ref[...].astype(jnp.bfloat16)
    h = res_ref[...] + jax.lax.dot_general(
        a, wo_ref[...].astype(jnp.bfloat16), (((1,), (1,)), ((), ())),
        preferred_element_type=jnp.float32)
    h_ref[...] = h
    var = jnp.mean(h * h, axis=1, keepdims=True)
    hn = ln_ref[...] * (h * jax.lax.rsqrt(var + _EPS))
    hn_ref[...] = hn
    logits = jax.lax.dot_general(hn, wg_ref[...], (((1,), (1,)), ((), ())),
                                 preferred_element_type=jnp.float32)
    e = logits.shape[1]
    lm = jnp.max(logits, axis=1, keepdims=True)
    ex = jnp.exp(logits - lm)
    p = ex / jnp.sum(ex, axis=1, keepdims=True)
    iota = jax.lax.broadcasted_iota(jnp.int32, p.shape, 1)
    m1 = jnp.max(p, axis=1, keepdims=True)
    i1 = jnp.min(jnp.where(p == m1, iota, e), axis=1, keepdims=True)
    sel1 = iota == i1
    p2 = jnp.where(sel1, -1.0, p)
    m2 = jnp.max(p2, axis=1, keepdims=True)
    i2 = jnp.min(jnp.where(p2 == m2, iota, e), axis=1, keepdims=True)
    sel2 = iota == i2
    denom = m1 + m2
    comb_ref[...] = (jnp.where(sel1, m1, 0.0) +
                     jnp.where(sel2, m2, 0.0)) / denom
    z = lm + jnp.log(jnp.sum(ex, axis=1, keepdims=True))
    tile_sum = jnp.sum(z * z, axis=0, keepdims=True)

    @pl.when(pl.program_id(0) == 0)
    def _():
        zacc_ref[...] = jnp.zeros_like(zacc_ref)

    zacc_ref[...] += tile_sum


def _moe_kernel(hn_ref, h_ref, comb_ref, wg_ref, wu_ref, wd_ref, o_ref):
    e = pl.program_id(1)
    f = pl.program_id(2)
    xs = hn_ref[...].astype(jnp.bfloat16)
    dims = (((1,), (1,)), ((), ()))
    g = jax.lax.dot_general(xs, wg_ref[0].astype(jnp.bfloat16), dims,
                            preferred_element_type=jnp.float32)
    u = jax.lax.dot_general(xs, wu_ref[0].astype(jnp.bfloat16), dims,
                            preferred_element_type=jnp.float32)
    a = (jax.nn.silu(g) * u).astype(jnp.bfloat16)
    y = jax.lax.dot_general(a, wd_ref[0].astype(jnp.bfloat16), dims,
                            preferred_element_type=jnp.float32)
    onehot = (jax.lax.broadcasted_iota(jnp.int32, (comb_ref.shape[1], 1), 0)
              == e).astype(jnp.float32)
    w = jnp.dot(comb_ref[...], onehot, preferred_element_type=jnp.float32)
    contrib = w * y

    @pl.when((e == 0) & (f == 0))
    def _():
        o_ref[...] = h_ref[...] + contrib

    @pl.when((e > 0) | (f > 0))
    def _():
        o_ref[...] += contrib


@jax.jit
def kernel(hidden_states, ln1_w, ln2_w, Wq, Wk, Wv, Wo, Wg, We_gate, We_up, We_down):
    B, S, D = hidden_states.shape
    E, FF, _ = We_gate.shape
    T = B * S
    TM = 512
    x = hidden_states.reshape(T, D)

    f32 = jnp.float32
    bf16 = jnp.bfloat16
    ln1 = ln1_w.reshape(1, D)
    ln2 = ln2_w.reshape(1, D)

    q, k, v = pl.pallas_call(
        _qkv_kernel,
        grid=(T // TM,),
        in_specs=[
            pl.BlockSpec((TM, D), lambda t: (t, 0)),
            pl.BlockSpec((1, D), lambda t: (0, 0)),
            pl.BlockSpec((D, D), lambda t: (0, 0)),
            pl.BlockSpec((D, D), lambda t: (0, 0)),
            pl.BlockSpec((D, D), lambda t: (0, 0)),
        ],
        out_specs=[
            pl.BlockSpec((TM, D), lambda t: (t, 0)),
            pl.BlockSpec((TM, D), lambda t: (t, 0)),
            pl.BlockSpec((TM, D), lambda t: (t, 0)),
        ],
        out_shape=[jax.ShapeDtypeStruct((T, D), f32)] * 3,
    )(x, ln1, Wq, Wk, Wv)

    def to_heads(t):
        return (t.reshape(B, S, _H, _DH).transpose(0, 2, 1, 3)
                .reshape(B * _H, S, _DH))

    qh, kh, vh = to_heads(q), to_heads(k), to_heads(v)

    inv_freq = 1.0 / (10000.0 ** (jnp.arange(0, _DH, 2, dtype=f32) / _DH))
    t_pos = jnp.arange(S, dtype=f32)
    freqs = jnp.outer(t_pos, inv_freq)
    emb = jnp.concatenate([freqs, freqs], axis=-1)
    cos = jnp.cos(emb)
    sin = jnp.sin(emb)

    attn = pl.pallas_call(
        _attn_kernel,
        grid=(B * _H,),
        in_specs=[
            pl.BlockSpec((1, S, _DH), lambda i: (i, 0, 0)),
            pl.BlockSpec((1, S, _DH), lambda i: (i, 0, 0)),
            pl.BlockSpec((1, S, _DH), lambda i: (i, 0, 0)),
            pl.BlockSpec((S, _DH), lambda i: (0, 0)),
            pl.BlockSpec((S, _DH), lambda i: (0, 0)),
        ],
        out_specs=pl.BlockSpec((1, S, _DH), lambda i: (i, 0, 0)),
        out_shape=jax.ShapeDtypeStruct((B * _H, S, _DH), f32),
    )(qh, kh, vh, cos, sin)

    attn_flat = (attn.reshape(B, _H, S, _DH).transpose(0, 2, 1, 3)
                 .reshape(T, D))

    h, hn, comb, zacc = pl.pallas_call(
        _post_attn_kernel,
        grid=(T // TM,),
        in_specs=[
            pl.BlockSpec((TM, D), lambda t: (t, 0)),
            pl.BlockSpec((TM, D), lambda t: (t, 0)),
            pl.BlockSpec((D, D), lambda t: (0, 0)),
            pl.BlockSpec((1, D), lambda t: (0, 0)),
            pl.BlockSpec((E, D), lambda t: (0, 0)),
        ],
        out_specs=[
            pl.BlockSpec((TM, D), lambda t: (t, 0)),
            pl.BlockSpec((TM, D), lambda t: (t, 0)),
            pl.BlockSpec((TM, E), lambda t: (t, 0)),
            pl.BlockSpec((1, 1), lambda t: (0, 0)),
        ],
        out_shape=[
            jax.ShapeDtypeStruct((T, D), f32),
            jax.ShapeDtypeStruct((T, D), f32),
            jax.ShapeDtypeStruct((T, E), f32),
            jax.ShapeDtypeStruct((1, 1), f32),
        ],
    )(attn_flat, x, Wo, ln2, Wg)

    aux_loss = _Z_LOSS_COEF * zacc[0, 0] / T

    TM2 = 1024
    FH = FF // 2
    out = pl.pallas_call(
        _moe_kernel,
        grid=(T // TM2, E, 2),
        in_specs=[
            pl.BlockSpec((TM2, D), lambda t, e, f: (t, 0)),
            pl.BlockSpec((TM2, D), lambda t, e, f: (t, 0)),
            pl.BlockSpec((TM2, E), lambda t, e, f: (t, 0)),
            pl.BlockSpec((1, FH, D), lambda t, e, f: (e, f, 0)),
            pl.BlockSpec((1, FH, D), lambda t, e, f: (e, f, 0)),
            pl.BlockSpec((1, D, FH), lambda t, e, f: (e, 0, f)),
        ],
        out_specs=pl.BlockSpec((TM2, D), lambda t, e, f: (t, 0)),
        out_shape=jax.ShapeDtypeStruct((T, D), f32),
    )(hn, h, comb, We_gate, We_up, We_down)

    return out.reshape(B, S, D), aux_loss
